# Initial kernel scaffold; baseline (speedup 1.0000x reference)
import jax
import jax.numpy as jnp
from jax.experimental import pallas as pl


def kernel(x, edge_index):
    raise NotImplementedError("write your pallas kernel here")


# speedup vs baseline: 5.7431x; 5.7431x over previous
import functools

import jax
import jax.numpy as jnp
from jax import lax
from jax.experimental import pallas as pl
from jax.experimental.pallas import tpu as pltpu
from jax.experimental.pallas import tpu_sc as plsc

D = 128
B = 64
NC = 2
NS = 16
NW = NC * NS


def _sc_scatter_gather(n_nodes, nblk):
  p_rows = -(-(n_nodes + 1) // (8 * NS)) * (8 * NS)
  rows_per_tile = p_rows // NS

  mesh = plsc.VectorSubcoreMesh(core_axis_name="c", subcore_axis_name="s")

  @functools.partial(
      pl.kernel,
      mesh=mesh,
      compiler_params=pltpu.CompilerParams(use_tc_tiling_on_sc=False),
      out_type=jax.ShapeDtypeStruct((NC, p_rows, D), jnp.float32),
      scratch_types=[
          pltpu.VMEM_SHARED((p_rows, D), jnp.float32),
          pltpu.VMEM((nblk, B), jnp.int32),
          pltpu.VMEM((nblk, B), jnp.int32),
          pltpu.VMEM((B, D), jnp.float32),
          pltpu.VMEM((B, D), jnp.float32),
          pltpu.SemaphoreType.DMA,
          pltpu.SemaphoreType.DMA,
      ],
  )
  def k(x_hbm, col_hbm, row_hbm, zero_hbm, out_hbm,
        accum, colb, rowb, rows0, rows1, gsem0, gsem1):
    c = lax.axis_index("c")
    s = lax.axis_index("s")
    wid = c * NS + s

    r0 = s * rows_per_tile
    pltpu.sync_copy(zero_hbm, accum.at[pl.ds(r0, rows_per_tile)])

    pltpu.sync_copy(col_hbm.at[wid], colb)
    pltpu.sync_copy(row_hbm.at[wid], rowb)

    plsc.subcore_barrier()

    bufs = (rows0, rows1)
    sems = (gsem0, gsem1)

    for b in range(2):
      pltpu.async_copy(x_hbm.at[colb.at[b]], bufs[b], sems[b])

    def body(i, carry):
      for b in range(2):
        g = i * 2 + b
        pltpu.make_async_copy(x_hbm.at[colb.at[g]], bufs[b], sems[b]).wait()
        pltpu.sync_copy(bufs[b], accum.at[rowb.at[g]], add=True)

        @pl.when(g + 2 < nblk)
        def _():
          pltpu.async_copy(x_hbm.at[colb.at[g + 2]], bufs[b], sems[b])
      return carry

    lax.fori_loop(0, nblk // 2, body, 0, unroll=False)

    plsc.subcore_barrier()

    pltpu.sync_copy(accum.at[pl.ds(r0, rows_per_tile)],
                    out_hbm.at[c, pl.ds(r0, rows_per_tile)])

  return k, p_rows


def _tc_combine(partials, n_nodes):
  blk = 1000

  def add_k(p_ref, o_ref):
    o_ref[...] = p_ref[0] + p_ref[1]

  return pl.pallas_call(
      add_k,
      grid=(n_nodes // blk,),
      in_specs=[pl.BlockSpec((2, blk, D), lambda i: (0, i, 0))],
      out_specs=pl.BlockSpec((blk, D), lambda i: (i, 0)),
      out_shape=jax.ShapeDtypeStruct((n_nodes, D), jnp.float32),
  )(partials)


@jax.jit
def kernel(x, edge_index):
  n_nodes = x.shape[0]
  e = edge_index.shape[1]
  row = edge_index[0].astype(jnp.int32)
  col = edge_index[1].astype(jnp.int32)

  nblk = -(-e // (NW * B))
  nblk += nblk % 2
  e_pad = NW * nblk * B
  pad = e_pad - e
  dummy_row = n_nodes
  row_p = jnp.concatenate([row, jnp.full((pad,), dummy_row, jnp.int32)])
  col_p = jnp.concatenate([col, jnp.zeros((pad,), jnp.int32)])
  row_b = row_p.reshape(NW, nblk, B)
  col_b = col_p.reshape(NW, nblk, B)

  sc_k, p_rows = _sc_scatter_gather(n_nodes, nblk)
  zeros = jnp.zeros((p_rows // NS, D), jnp.float32)
  partials = sc_k(x, col_b, row_b, zeros)
  return _tc_combine(partials, n_nodes)


# speedup vs baseline: 11.1190x; 1.9361x over previous
import functools

import jax
import jax.numpy as jnp
from jax import lax
from jax.experimental import pallas as pl
from jax.experimental.pallas import tpu as pltpu
from jax.experimental.pallas import tpu_sc as plsc

D = 128
B = 64
NC = 2
NS = 16
NW = NC * NS


def _sc_scatter_gather(n_nodes, nblk):
  p_rows = -(-(n_nodes + 1) // (8 * NS)) * (8 * NS)
  rows_per_tile = p_rows // NS

  mesh = plsc.VectorSubcoreMesh(core_axis_name="c", subcore_axis_name="s")

  @functools.partial(
      pl.kernel,
      mesh=mesh,
      compiler_params=pltpu.CompilerParams(use_tc_tiling_on_sc=False),
      out_type=jax.ShapeDtypeStruct((NC, p_rows, D), jnp.float32),
      scratch_types=[
          pltpu.VMEM_SHARED((p_rows, D), jnp.float32),
          pltpu.VMEM((nblk, B), jnp.int32),
          pltpu.VMEM((nblk, B), jnp.int32),
          pltpu.VMEM((B, D), jnp.float32),
          pltpu.VMEM((B, D), jnp.float32),
          pltpu.SemaphoreType.DMA,
          pltpu.SemaphoreType.DMA,
      ],
  )
  def k(x_hbm, col_hbm, row_hbm, zero_hbm, out_hbm,
        accum, colb, rowb, rows0, rows1, gsem0, gsem1):
    c = lax.axis_index("c")
    s = lax.axis_index("s")
    wid = c * NS + s

    r0 = s * rows_per_tile
    pltpu.sync_copy(zero_hbm, accum.at[pl.ds(r0, rows_per_tile)])

    pltpu.sync_copy(col_hbm.at[wid], colb)
    pltpu.sync_copy(row_hbm.at[wid], rowb)

    plsc.subcore_barrier()

    bufs = (rows0, rows1)
    sems = (gsem0, gsem1)

    for b in range(2):
      pltpu.async_copy(x_hbm.at[colb.at[b]], bufs[b], sems[b])

    def body(i, carry):
      for b in range(2):
        g = i * 2 + b
        pltpu.make_async_copy(x_hbm.at[colb.at[g]], bufs[b], sems[b]).wait()
        pltpu.sync_copy(bufs[b], accum.at[rowb.at[g]], add=True)

        @pl.when(g + 2 < nblk)
        def _():
          pltpu.async_copy(x_hbm.at[colb.at[g + 2]], bufs[b], sems[b])
      return carry

    lax.fori_loop(0, nblk // 2, body, 0, unroll=False)

    plsc.subcore_barrier()

    pltpu.sync_copy(accum.at[pl.ds(r0, rows_per_tile)],
                    out_hbm.at[c, pl.ds(r0, rows_per_tile)])

  return k, p_rows


def _tc_combine(partials, n_nodes):
  blk = 1000

  def add_k(p_ref, o_ref):
    o_ref[...] = p_ref[0] + p_ref[1]

  return pl.pallas_call(
      add_k,
      grid=(n_nodes // blk,),
      in_specs=[pl.BlockSpec((2, blk, D), lambda i: (0, i, 0))],
      out_specs=pl.BlockSpec((blk, D), lambda i: (i, 0)),
      out_shape=jax.ShapeDtypeStruct((n_nodes, D), jnp.float32),
  )(partials)


@jax.jit
def kernel(x, edge_index):
  n_nodes = x.shape[0]
  e = edge_index.shape[1]
  row = edge_index[0].astype(jnp.int32)
  col = edge_index[1].astype(jnp.int32)

  nblk = -(-e // (NW * B))
  nblk += nblk % 2
  e_pad = NW * nblk * B
  pad = e_pad - e
  p_rows = -(-(n_nodes + 1) // (8 * NS)) * (8 * NS)
  pad_idx = jnp.arange(pad, dtype=jnp.int32)
  row_p = jnp.concatenate([row, n_nodes + pad_idx % (p_rows - n_nodes)])
  col_p = jnp.concatenate([col, pad_idx % n_nodes])
  row_b = row_p.reshape(NW, nblk, B)
  col_b = col_p.reshape(NW, nblk, B)

  sc_k, p_rows = _sc_scatter_gather(n_nodes, nblk)
  zeros = jnp.zeros((p_rows // NS, D), jnp.float32)
  partials = sc_k(x, col_b, row_b, zeros)
  return _tc_combine(partials, n_nodes)


# speedup vs baseline: 12.4872x; 1.1231x over previous
import functools

import jax
import jax.numpy as jnp
from jax import lax
from jax.experimental import pallas as pl
from jax.experimental.pallas import tpu as pltpu
from jax.experimental.pallas import tpu_sc as plsc

D = 128
B = 128
NSTAGE = 2
NC = 2
NS = 16
NW = NC * NS


def _sc_scatter_gather(n_nodes, nblk):
  p_rows = -(-(n_nodes + 1) // (8 * NS)) * (8 * NS)
  rows_per_tile = p_rows // NS
  hb = nblk // NSTAGE

  mesh = plsc.VectorSubcoreMesh(core_axis_name="c", subcore_axis_name="s")

  @functools.partial(
      pl.kernel,
      mesh=mesh,
      compiler_params=pltpu.CompilerParams(use_tc_tiling_on_sc=False),
      out_type=jax.ShapeDtypeStruct((NC, p_rows, D), jnp.float32),
      scratch_types=[
          pltpu.VMEM_SHARED((p_rows, D), jnp.float32),
          pltpu.VMEM((hb, B), jnp.int32),
          pltpu.VMEM((hb, B), jnp.int32),
          pltpu.VMEM((B, D), jnp.float32),
          pltpu.VMEM((B, D), jnp.float32),
          pltpu.SemaphoreType.DMA,
          pltpu.SemaphoreType.DMA,
      ],
  )
  def k(x_hbm, col_hbm, row_hbm, zero_hbm, out_hbm,
        accum, colb, rowb, rows0, rows1, gsem0, gsem1):
    c = lax.axis_index("c")
    s = lax.axis_index("s")
    wid = c * NS + s

    r0 = s * rows_per_tile
    pltpu.sync_copy(zero_hbm, accum.at[pl.ds(r0, rows_per_tile)])

    plsc.subcore_barrier()

    bufs = (rows0, rows1)
    sems = (gsem0, gsem1)

    for st in range(NSTAGE):
      pltpu.sync_copy(col_hbm.at[wid, pl.ds(st * hb, hb)], colb)
      pltpu.sync_copy(row_hbm.at[wid, pl.ds(st * hb, hb)], rowb)

      for b in range(2):
        pltpu.async_copy(x_hbm.at[colb.at[b]], bufs[b], sems[b])

      def body(i, carry):
        for b in range(2):
          g = i * 2 + b
          pltpu.make_async_copy(x_hbm.at[colb.at[g]], bufs[b], sems[b]).wait()
          pltpu.sync_copy(bufs[b], accum.at[rowb.at[g]], add=True)

          @pl.when(g + 2 < hb)
          def _():
            pltpu.async_copy(x_hbm.at[colb.at[g + 2]], bufs[b], sems[b])
        return carry

      lax.fori_loop(0, hb // 2, body, 0, unroll=False)

    plsc.subcore_barrier()

    pltpu.sync_copy(accum.at[pl.ds(r0, rows_per_tile)],
                    out_hbm.at[c, pl.ds(r0, rows_per_tile)])

  return k, p_rows


def _tc_combine(partials, n_nodes):
  blk = 1000

  def add_k(p_ref, o_ref):
    o_ref[...] = p_ref[0] + p_ref[1]

  return pl.pallas_call(
      add_k,
      grid=(n_nodes // blk,),
      in_specs=[pl.BlockSpec((2, blk, D), lambda i: (0, i, 0))],
      out_specs=pl.BlockSpec((blk, D), lambda i: (i, 0)),
      out_shape=jax.ShapeDtypeStruct((n_nodes, D), jnp.float32),
  )(partials)


@jax.jit
def kernel(x, edge_index):
  n_nodes = x.shape[0]
  e = edge_index.shape[1]
  row = edge_index[0].astype(jnp.int32)
  col = edge_index[1].astype(jnp.int32)

  nblk = -(-e // (NW * B))
  m = 2 * NSTAGE
  nblk = -(-nblk // m) * m
  e_pad = NW * nblk * B
  pad = e_pad - e
  p_rows = -(-(n_nodes + 1) // (8 * NS)) * (8 * NS)
  pad_idx = jnp.arange(pad, dtype=jnp.int32)
  row_p = jnp.concatenate([row, n_nodes + pad_idx % (p_rows - n_nodes)])
  col_p = jnp.concatenate([col, pad_idx % n_nodes])
  row_b = row_p.reshape(NW, nblk, B)
  col_b = col_p.reshape(NW, nblk, B)

  sc_k, p_rows = _sc_scatter_gather(n_nodes, nblk)
  zeros = jnp.zeros((p_rows // NS, D), jnp.float32)
  partials = sc_k(x, col_b, row_b, zeros)
  return _tc_combine(partials, n_nodes)


# speedup vs baseline: 12.5327x; 1.0036x over previous
import functools

import jax
import jax.numpy as jnp
from jax import lax
from jax.experimental import pallas as pl
from jax.experimental.pallas import tpu as pltpu
from jax.experimental.pallas import tpu_sc as plsc

D = 128
B = 128
NSTAGE = 2
NC = 2
NS = 16
NW = NC * NS


def _sc_scatter_gather(n_nodes, nblk):
  p_rows = -(-(n_nodes + 1) // (8 * NS)) * (8 * NS)
  rows_per_tile = p_rows // NS
  hb = nblk // NSTAGE

  mesh = plsc.VectorSubcoreMesh(core_axis_name="c", subcore_axis_name="s")

  @functools.partial(
      pl.kernel,
      mesh=mesh,
      compiler_params=pltpu.CompilerParams(use_tc_tiling_on_sc=True),
      out_type=jax.ShapeDtypeStruct((NC, p_rows, D), jnp.float32),
      scratch_types=[
          pltpu.VMEM_SHARED((p_rows, D), jnp.float32),
          pltpu.VMEM((hb, B), jnp.int32),
          pltpu.VMEM((hb, B), jnp.int32),
          pltpu.VMEM((B, D), jnp.float32),
          pltpu.VMEM((B, D), jnp.float32),
          pltpu.SemaphoreType.DMA,
          pltpu.SemaphoreType.DMA,
      ],
  )
  def k(x_hbm, col_hbm, row_hbm, zero_hbm, out_hbm,
        accum, colb, rowb, rows0, rows1, gsem0, gsem1):
    c = lax.axis_index("c")
    s = lax.axis_index("s")
    wid = c * NS + s

    r0 = s * rows_per_tile
    pltpu.sync_copy(zero_hbm, accum.at[pl.ds(r0, rows_per_tile)])

    plsc.subcore_barrier()

    bufs = (rows0, rows1)
    sems = (gsem0, gsem1)

    for st in range(NSTAGE):
      pltpu.sync_copy(col_hbm.at[wid, pl.ds(st * hb, hb)], colb)
      pltpu.sync_copy(row_hbm.at[wid, pl.ds(st * hb, hb)], rowb)

      for b in range(2):
        pltpu.async_copy(x_hbm.at[colb.at[b]], bufs[b], sems[b])

      def body(i, carry):
        for b in range(2):
          g = i * 2 + b
          pltpu.make_async_copy(x_hbm.at[colb.at[g]], bufs[b], sems[b]).wait()
          pltpu.sync_copy(bufs[b], accum.at[rowb.at[g]], add=True)

          @pl.when(g + 2 < hb)
          def _():
            pltpu.async_copy(x_hbm.at[colb.at[g + 2]], bufs[b], sems[b])
        return carry

      lax.fori_loop(0, hb // 2, body, 0, unroll=False)

    plsc.subcore_barrier()

    pltpu.sync_copy(accum.at[pl.ds(r0, rows_per_tile)],
                    out_hbm.at[c, pl.ds(r0, rows_per_tile)])

  return k, p_rows


def _tc_combine(partials, n_nodes):
  blk = 1000

  def add_k(p_ref, o_ref):
    o_ref[...] = p_ref[0] + p_ref[1]

  return pl.pallas_call(
      add_k,
      grid=(n_nodes // blk,),
      in_specs=[pl.BlockSpec((2, blk, D), lambda i: (0, i, 0))],
      out_specs=pl.BlockSpec((blk, D), lambda i: (i, 0)),
      out_shape=jax.ShapeDtypeStruct((n_nodes, D), jnp.float32),
  )(partials)


@jax.jit
def kernel(x, edge_index):
  n_nodes = x.shape[0]
  e = edge_index.shape[1]
  row = edge_index[0].astype(jnp.int32)
  col = edge_index[1].astype(jnp.int32)

  nblk = -(-e // (NW * B))
  m = 2 * NSTAGE
  nblk = -(-nblk // m) * m
  e_pad = NW * nblk * B
  pad = e_pad - e
  p_rows = -(-(n_nodes + 1) // (8 * NS)) * (8 * NS)
  pad_idx = jnp.arange(pad, dtype=jnp.int32)
  row_p = jnp.concatenate([row, n_nodes + pad_idx % (p_rows - n_nodes)])
  col_p = jnp.concatenate([col, pad_idx % n_nodes])
  row_b = row_p.reshape(NW, nblk, B)
  col_b = col_p.reshape(NW, nblk, B)

  sc_k, p_rows = _sc_scatter_gather(n_nodes, nblk)
  zeros = jnp.zeros((p_rows // NS, D), jnp.float32)
  partials = sc_k(x, col_b, row_b, zeros)
  return _tc_combine(partials, n_nodes)


# speedup vs baseline: 12.8886x; 1.0284x over previous
import functools

import jax
import jax.numpy as jnp
import numpy as np
from jax import lax
from jax.experimental import pallas as pl
from jax.experimental.pallas import tpu as pltpu
from jax.experimental.pallas import tpu_sc as plsc

D = 128
B = 128
NC = 2
NS = 16
NW = NC * NS


def _sc_scatter_gather(n_nodes, nblk):
  p_rows = -(-(n_nodes + 1) // (8 * NS)) * (8 * NS)
  rows_per_tile = p_rows // NS
  epw = nblk * B

  mesh = plsc.VectorSubcoreMesh(core_axis_name="c", subcore_axis_name="s")

  @functools.partial(
      pl.kernel,
      mesh=mesh,
      compiler_params=pltpu.CompilerParams(use_tc_tiling_on_sc=True),
      out_type=jax.ShapeDtypeStruct((NC, p_rows, D), jnp.float32),
      scratch_types=[
          pltpu.VMEM_SHARED((p_rows, D), jnp.float32),
          pltpu.VMEM((epw,), jnp.int32),
          pltpu.VMEM((B,), jnp.int32),
          pltpu.VMEM((B,), jnp.int32),
          pltpu.VMEM((B, D), jnp.float32),
          pltpu.VMEM((B, D), jnp.float32),
          pltpu.SemaphoreType.DMA,
          pltpu.SemaphoreType.DMA,
          pltpu.SemaphoreType.DMA,
          pltpu.SemaphoreType.DMA,
      ],
  )
  def k(x_hbm, col_hbm, row_hbm, zero_hbm, out_hbm,
        accum, colb, rs0, rs1, rows0, rows1, gsem0, gsem1, rsem0, rsem1):
    c = lax.axis_index("c")
    s = lax.axis_index("s")
    wid = c * NS + s
    e0 = wid * epw

    rslots = (rs0, rs1)
    rsems = (rsem0, rsem1)
    bufs = (rows0, rows1)
    gsems = (gsem0, gsem1)

    pltpu.sync_copy(col_hbm.at[pl.ds(e0, epw)], colb)
    for b in range(2):
      pltpu.async_copy(row_hbm.at[pl.ds(e0 + b * B, B)], rslots[b], rsems[b])
      pltpu.async_copy(x_hbm.at[colb.at[pl.ds(b * B, B)]], bufs[b], gsems[b])

    r0 = s * rows_per_tile
    pltpu.sync_copy(zero_hbm, accum.at[pl.ds(r0, rows_per_tile)])

    plsc.subcore_barrier()

    def body(i, carry):
      for b in range(2):
        g = i * 2 + b
        pltpu.make_async_copy(
            x_hbm.at[colb.at[pl.ds(g * B, B)]], bufs[b], gsems[b]).wait()
        pltpu.make_async_copy(
            row_hbm.at[pl.ds(e0 + g * B, B)], rslots[b], rsems[b]).wait()
        pltpu.sync_copy(bufs[b], accum.at[rslots[b]], add=True)

        @pl.when(g + 2 < nblk)
        def _():
          g2 = g + 2
          pltpu.async_copy(
              row_hbm.at[pl.ds(e0 + g2 * B, B)], rslots[b], rsems[b])
          pltpu.async_copy(
              x_hbm.at[colb.at[pl.ds(g2 * B, B)]], bufs[b], gsems[b])
      return carry

    lax.fori_loop(0, nblk // 2, body, 0, unroll=False)

    plsc.subcore_barrier()

    pltpu.sync_copy(accum.at[pl.ds(r0, rows_per_tile)],
                    out_hbm.at[c, pl.ds(r0, rows_per_tile)])

  return k, p_rows


def _tc_combine(partials, n_nodes):
  blk = 1000

  def add_k(p_ref, o_ref):
    o_ref[...] = p_ref[0] + p_ref[1]

  return pl.pallas_call(
      add_k,
      grid=(n_nodes // blk,),
      in_specs=[pl.BlockSpec((2, blk, D), lambda i: (0, i, 0))],
      out_specs=pl.BlockSpec((blk, D), lambda i: (i, 0)),
      out_shape=jax.ShapeDtypeStruct((n_nodes, D), jnp.float32),
  )(partials)


@jax.jit
def kernel(x, edge_index):
  n_nodes = x.shape[0]
  e = edge_index.shape[1]
  row = edge_index[0].astype(jnp.int32)
  col = edge_index[1].astype(jnp.int32)

  nblk = -(-e // (NW * B))
  nblk += nblk % 2
  e_pad = NW * nblk * B
  pad = e_pad - e
  p_rows = -(-(n_nodes + 1) // (8 * NS)) * (8 * NS)
  pad_idx = np.arange(pad, dtype=np.int32)
  row_pad = jnp.asarray(n_nodes + pad_idx % (p_rows - n_nodes))
  col_pad = jnp.asarray(pad_idx % n_nodes)
  row_p = jnp.concatenate([row, row_pad])
  col_p = jnp.concatenate([col, col_pad])

  sc_k, p_rows2 = _sc_scatter_gather(n_nodes, nblk)
  assert p_rows2 == p_rows
  zeros = jnp.zeros((p_rows // NS, D), jnp.float32)
  partials = sc_k(x, col_p, row_p, zeros)
  return _tc_combine(partials, n_nodes)


# speedup vs baseline: 14.2597x; 1.1064x over previous
import functools

import jax
import jax.numpy as jnp
import numpy as np
from jax import lax
from jax.experimental import pallas as pl
from jax.experimental.pallas import tpu as pltpu
from jax.experimental.pallas import tpu_sc as plsc

D = 128
B = 128
NC = 2
NS = 16
NW = NC * NS


def _sc_scatter_gather(n_nodes, n_edges, nblk):
  p_rows = -(-(n_nodes + 1) // (8 * NS)) * (8 * NS)
  rows_per_tile = p_rows // NS
  epw = nblk * B
  e = n_edges
  w_str = e // epw
  m_str = e - w_str * epw
  assert e % B == 0 and m_str % 8 == 0 and w_str == NW - 1

  mesh = plsc.VectorSubcoreMesh(core_axis_name="c", subcore_axis_name="s")

  @functools.partial(
      pl.kernel,
      mesh=mesh,
      compiler_params=pltpu.CompilerParams(use_tc_tiling_on_sc=True),
      out_type=jax.ShapeDtypeStruct((NC, p_rows, D), jnp.float32),
      scratch_types=[
          pltpu.VMEM_SHARED((p_rows, D), jnp.float32),
          pltpu.VMEM((epw,), jnp.int32),
          pltpu.VMEM((B,), jnp.int32),
          pltpu.VMEM((B,), jnp.int32),
          pltpu.VMEM((B, D), jnp.float32),
          pltpu.VMEM((B, D), jnp.float32),
          pltpu.SemaphoreType.DMA,
          pltpu.SemaphoreType.DMA,
          pltpu.SemaphoreType.DMA,
          pltpu.SemaphoreType.DMA,
      ],
  )
  def k(x_hbm, edge_hbm, rowpad_hbm, colpad_hbm, zero_hbm, out_hbm,
        accum, colb, rs0, rs1, rows0, rows1, gsem0, gsem1, rsem0, rsem1):
    c = lax.axis_index("c")
    s = lax.axis_index("s")
    wid = c * NS + s
    e0 = wid * epw

    rslots = (rs0, rs1)
    rsems = (rsem0, rsem1)
    bufs = (rows0, rows1)
    gsems = (gsem0, gsem1)

    @pl.when(wid < w_str)
    def _():
      pltpu.sync_copy(edge_hbm.at[1, pl.ds(e0, epw)], colb)

    @pl.when(wid == w_str)
    def _():
      pltpu.sync_copy(edge_hbm.at[1, pl.ds(w_str * epw, m_str)],
                      colb.at[pl.ds(0, m_str)])
      pltpu.sync_copy(colpad_hbm, colb.at[pl.ds(m_str, epw - m_str)])

    def stage_rows(g, slot, sem):
      start = e0 + g * B
      @pl.when(start + B <= e)
      def _():
        pltpu.async_copy(edge_hbm.at[0, pl.ds(start, B)], slot, sem)

      @pl.when(start >= e)
      def _():
        pltpu.async_copy(rowpad_hbm.at[pl.ds(start - e, B)], slot, sem)

    for b in range(2):
      stage_rows(b, rslots[b], rsems[b])
      pltpu.async_copy(x_hbm.at[colb.at[pl.ds(b * B, B)]], bufs[b], gsems[b])

    r0 = s * rows_per_tile
    pltpu.sync_copy(zero_hbm, accum.at[pl.ds(r0, rows_per_tile)])

    plsc.subcore_barrier()

    def body(i, carry):
      for b in range(2):
        g = i * 2 + b
        pltpu.make_async_copy(
            x_hbm.at[colb.at[pl.ds(g * B, B)]], bufs[b], gsems[b]).wait()
        pltpu.make_async_copy(
            rowpad_hbm.at[pl.ds(0, B)], rslots[b], rsems[b]).wait()
        pltpu.sync_copy(bufs[b], accum.at[rslots[b]], add=True)

        @pl.when(g + 2 < nblk)
        def _():
          g2 = g + 2
          stage_rows(g2, rslots[b], rsems[b])
          pltpu.async_copy(
              x_hbm.at[colb.at[pl.ds(g2 * B, B)]], bufs[b], gsems[b])
      return carry

    lax.fori_loop(0, nblk // 2, body, 0, unroll=False)

    plsc.subcore_barrier()

    pltpu.sync_copy(accum.at[pl.ds(r0, rows_per_tile)],
                    out_hbm.at[c, pl.ds(r0, rows_per_tile)])

  return k, p_rows


def _tc_combine(partials, n_nodes):
  blk = 1000

  def add_k(p_ref, o_ref):
    o_ref[...] = p_ref[0] + p_ref[1]

  return pl.pallas_call(
      add_k,
      grid=(n_nodes // blk,),
      in_specs=[pl.BlockSpec((2, blk, D), lambda i: (0, i, 0))],
      out_specs=pl.BlockSpec((blk, D), lambda i: (i, 0)),
      out_shape=jax.ShapeDtypeStruct((n_nodes, D), jnp.float32),
  )(partials)


@jax.jit
def kernel(x, edge_index):
  n_nodes = x.shape[0]
  e = edge_index.shape[1]
  edge32 = edge_index.astype(jnp.int32)

  nblk = -(-e // (NW * B))
  nblk += nblk % 2
  e_pad = NW * nblk * B
  pad = e_pad - e
  p_rows = -(-(n_nodes + 1) // (8 * NS)) * (8 * NS)
  pad_idx = np.arange(pad, dtype=np.int32)
  row_pad = jnp.asarray(n_nodes + pad_idx % (p_rows - n_nodes))
  col_pad = jnp.asarray(pad_idx % n_nodes)

  sc_k, p_rows2 = _sc_scatter_gather(n_nodes, e, nblk)
  assert p_rows2 == p_rows
  zeros = jnp.zeros((p_rows // NS, D), jnp.float32)
  partials = sc_k(x, edge32, row_pad, col_pad, zeros)
  return _tc_combine(partials, n_nodes)


# speedup vs baseline: 15.3010x; 1.0730x over previous
import functools

import jax
import jax.numpy as jnp
import numpy as np
from jax import lax
from jax.experimental import pallas as pl
from jax.experimental.pallas import tpu as pltpu
from jax.experimental.pallas import tpu_sc as plsc

D = 128
B = 96
NBUF = 3
NC = 2
NS = 16
NW = NC * NS


def _sc_scatter_gather(n_nodes, n_edges, nblk):
  p_rows = -(-(n_nodes + 1) // (8 * NS)) * (8 * NS)
  rows_per_tile = p_rows // NS
  epw = nblk * B
  e = n_edges
  w_str = e // epw
  assert w_str >= 1 and (e - w_str * epw) % 8 == 0 and e % 8 == 0

  mesh = plsc.VectorSubcoreMesh(core_axis_name="c", subcore_axis_name="s")

  @functools.partial(
      pl.kernel,
      mesh=mesh,
      compiler_params=pltpu.CompilerParams(use_tc_tiling_on_sc=False),
      out_type=jax.ShapeDtypeStruct((NC, p_rows, D), jnp.float32),
      scratch_types=[
          pltpu.VMEM_SHARED((p_rows, D), jnp.float32),
          pltpu.VMEM((epw,), jnp.int32),
      ] + [pltpu.VMEM((B,), jnp.int32) for _ in range(NBUF)]
        + [pltpu.VMEM((B, D), jnp.float32) for _ in range(NBUF)]
        + [pltpu.SemaphoreType.DMA] * (2 * NBUF),
  )
  def k(x_hbm, edge_hbm, rowpad_hbm, colpad_hbm, zero_hbm, out_hbm,
        accum, colb, *bufs_and_sems):
    rslots = bufs_and_sems[:NBUF]
    bufs = bufs_and_sems[NBUF:2 * NBUF]
    gsems = bufs_and_sems[2 * NBUF:3 * NBUF]
    rsems = bufs_and_sems[3 * NBUF:4 * NBUF]

    c = lax.axis_index("c")
    s = lax.axis_index("s")
    wid = c * NS + s
    e0 = wid * epw

    @pl.when(wid < w_str)
    def _():
      pltpu.sync_copy(edge_hbm.at[1, pl.ds(e0, epw)], colb)

    for w in range(w_str, NW):
      @pl.when(wid == w)
      def _(w=w):
        ms = min(max(e - w * epw, 0), epw)
        if ms:
          pltpu.sync_copy(edge_hbm.at[1, pl.ds(w * epw, ms)],
                          colb.at[pl.ds(0, ms)])
        po = w * epw + ms - e
        pltpu.sync_copy(colpad_hbm.at[pl.ds(po, epw - ms)],
                        colb.at[pl.ds(ms, epw - ms)])

    def stage_rows(g, slot, sem):
      start = e0 + g * B
      rem = e % B

      @pl.when(start + B <= e)
      def _():
        pltpu.async_copy(edge_hbm.at[0, pl.ds(start, B)], slot, sem)

      @pl.when(start >= e)
      def _():
        pltpu.async_copy(rowpad_hbm.at[pl.ds(start - e, B)], slot, sem)

      if rem:
        @pl.when(jnp.logical_and(start < e, start + B > e))
        def _():
          pltpu.async_copy(edge_hbm.at[0, pl.ds(e - rem, rem)],
                           slot.at[pl.ds(0, rem)], sem)
          pltpu.async_copy(rowpad_hbm.at[pl.ds(0, B - rem)],
                           slot.at[pl.ds(rem, B - rem)], sem)

    for b in range(NBUF):
      stage_rows(b, rslots[b], rsems[b])
      pltpu.async_copy(x_hbm.at[colb.at[pl.ds(b * B, B)]], bufs[b], gsems[b])

    r0 = s * rows_per_tile
    pltpu.sync_copy(zero_hbm, accum.at[pl.ds(r0, rows_per_tile)])

    plsc.subcore_barrier()

    def body(i, carry):
      for b in range(NBUF):
        g = i * NBUF + b
        pltpu.make_async_copy(
            x_hbm.at[colb.at[pl.ds(g * B, B)]], bufs[b], gsems[b]).wait()
        pltpu.make_async_copy(
            rowpad_hbm.at[pl.ds(0, B)], rslots[b], rsems[b]).wait()
        pltpu.sync_copy(bufs[b], accum.at[rslots[b]], add=True)

        @pl.when(g + NBUF < nblk)
        def _():
          g2 = g + NBUF
          stage_rows(g2, rslots[b], rsems[b])
          pltpu.async_copy(
              x_hbm.at[colb.at[pl.ds(g2 * B, B)]], bufs[b], gsems[b])
      return carry

    lax.fori_loop(0, nblk // NBUF, body, 0, unroll=False)

    plsc.subcore_barrier()

    pltpu.sync_copy(accum.at[pl.ds(r0, rows_per_tile)],
                    out_hbm.at[c, pl.ds(r0, rows_per_tile)])

  return k, p_rows


def _tc_combine(partials, n_nodes):
  blk = 1000

  def add_k(p_ref, o_ref):
    o_ref[...] = p_ref[0] + p_ref[1]

  return pl.pallas_call(
      add_k,
      grid=(n_nodes // blk,),
      in_specs=[pl.BlockSpec((2, blk, D), lambda i: (0, i, 0))],
      out_specs=pl.BlockSpec((blk, D), lambda i: (i, 0)),
      out_shape=jax.ShapeDtypeStruct((n_nodes, D), jnp.float32),
  )(partials)


@jax.jit
def kernel(x, edge_index):
  n_nodes = x.shape[0]
  e = edge_index.shape[1]
  edge32 = edge_index.astype(jnp.int32)

  nblk = -(-e // (NW * B))
  nblk = -(-nblk // NBUF) * NBUF
  e_pad = NW * nblk * B
  pad = e_pad - e
  p_rows = -(-(n_nodes + 1) // (8 * NS)) * (8 * NS)
  pad_idx = np.arange(max(pad, 1), dtype=np.int32)
  row_pad = jnp.asarray(n_nodes + pad_idx % (p_rows - n_nodes))
  col_pad = jnp.asarray(pad_idx % n_nodes)

  sc_k, p_rows2 = _sc_scatter_gather(n_nodes, e, nblk)
  assert p_rows2 == p_rows
  zeros = jnp.zeros((p_rows // NS, D), jnp.float32)
  partials = sc_k(x, edge32, row_pad, col_pad, zeros)
  return _tc_combine(partials, n_nodes)


# speedup vs baseline: 15.5838x; 1.0185x over previous
import functools

import jax
import jax.numpy as jnp
import numpy as np
from jax import lax
from jax.experimental import pallas as pl
from jax.experimental.pallas import tpu as pltpu
from jax.experimental.pallas import tpu_sc as plsc

D = 128
B = 96
NBUF = 3
NC = 2
NS = 16
NW = NC * NS


def _sc_scatter_gather(n_nodes, n_edges, nblk):
  p_rows = -(-(n_nodes + 1) // (8 * NS)) * (8 * NS)
  rows_per_tile = p_rows // NS
  epw = nblk * B
  e = n_edges
  w_str = e // epw
  assert w_str >= 1 and (e - w_str * epw) % 8 == 0 and e % 8 == 0

  mesh = plsc.VectorSubcoreMesh(core_axis_name="c", subcore_axis_name="s")

  @functools.partial(
      pl.kernel,
      mesh=mesh,
      compiler_params=pltpu.CompilerParams(use_tc_tiling_on_sc=False),
      out_type=jax.ShapeDtypeStruct((NC, p_rows, D), jnp.float32),
      scratch_types=[
          pltpu.VMEM_SHARED((p_rows, D), jnp.float32),
          pltpu.VMEM((epw,), jnp.int32),
      ] + [pltpu.VMEM((B,), jnp.int32) for _ in range(NBUF)]
        + [pltpu.VMEM((B, D), jnp.float32) for _ in range(NBUF)]
        + [pltpu.SemaphoreType.DMA] * (2 * NBUF),
  )
  def k(x_hbm, edge_hbm, rowpad_hbm, colpad_hbm, zero_hbm, out_hbm,
        accum, colb, *bufs_and_sems):
    rslots = bufs_and_sems[:NBUF]
    bufs = bufs_and_sems[NBUF:2 * NBUF]
    gsems = bufs_and_sems[2 * NBUF:3 * NBUF]
    rsems = bufs_and_sems[3 * NBUF:4 * NBUF]

    c = lax.axis_index("c")
    s = lax.axis_index("s")
    wid = c * NS + s
    e0 = wid * epw

    @pl.when(wid < w_str)
    def _():
      pltpu.sync_copy(edge_hbm.at[1, pl.ds(e0, epw)], colb)

    for w in range(w_str, NW):
      @pl.when(wid == w)
      def _(w=w):
        ms = min(max(e - w * epw, 0), epw)
        if ms:
          pltpu.sync_copy(edge_hbm.at[1, pl.ds(w * epw, ms)],
                          colb.at[pl.ds(0, ms)])
        po = w * epw + ms - e
        pltpu.sync_copy(colpad_hbm.at[pl.ds(po, epw - ms)],
                        colb.at[pl.ds(ms, epw - ms)])

    def stage_rows(g, slot, sem):
      start = e0 + g * B
      rem = e % B

      @pl.when(start + B <= e)
      def _():
        pltpu.async_copy(edge_hbm.at[0, pl.ds(start, B)], slot, sem)

      @pl.when(start >= e)
      def _():
        pltpu.async_copy(rowpad_hbm.at[pl.ds(start - e, B)], slot, sem)

      if rem:
        @pl.when(jnp.logical_and(start < e, start + B > e))
        def _():
          pltpu.async_copy(edge_hbm.at[0, pl.ds(e - rem, rem)],
                           slot.at[pl.ds(0, rem)], sem)
          pltpu.async_copy(rowpad_hbm.at[pl.ds(0, B - rem)],
                           slot.at[pl.ds(rem, B - rem)], sem)

    for b in range(NBUF):
      stage_rows(b, rslots[b], rsems[b])
      pltpu.async_copy(x_hbm.at[colb.at[pl.ds(b * B, B)]], bufs[b], gsems[b])

    r0 = s * rows_per_tile
    pltpu.sync_copy(zero_hbm, accum.at[pl.ds(r0, rows_per_tile)])

    plsc.subcore_barrier()

    def body(i, carry):
      for b in range(NBUF):
        g = i * NBUF + b
        pltpu.make_async_copy(
            x_hbm.at[colb.at[pl.ds(g * B, B)]], bufs[b], gsems[b]).wait()
        pltpu.make_async_copy(
            rowpad_hbm.at[pl.ds(0, B)], rslots[b], rsems[b]).wait()
        pltpu.sync_copy(bufs[b], accum.at[rslots[b]], add=True)

        @pl.when(g + NBUF < nblk)
        def _():
          g2 = g + NBUF
          stage_rows(g2, rslots[b], rsems[b])
          pltpu.async_copy(
              x_hbm.at[colb.at[pl.ds(g2 * B, B)]], bufs[b], gsems[b])
      return carry

    lax.fori_loop(0, nblk // NBUF, body, 0, unroll=False)

    plsc.subcore_barrier()

    pltpu.sync_copy(accum.at[pl.ds(r0, rows_per_tile)],
                    out_hbm.at[c, pl.ds(r0, rows_per_tile)])

  return k, p_rows


def _tc_combine(partials, n_nodes):
  blk = 2000

  def add_k(p_ref, o_ref):
    o_ref[...] = p_ref[0] + p_ref[1]

  return pl.pallas_call(
      add_k,
      grid=(n_nodes // blk,),
      in_specs=[pl.BlockSpec((2, blk, D), lambda i: (0, i, 0))],
      out_specs=pl.BlockSpec((blk, D), lambda i: (i, 0)),
      out_shape=jax.ShapeDtypeStruct((n_nodes, D), jnp.float32),
  )(partials)


@jax.jit
def kernel(x, edge_index):
  n_nodes = x.shape[0]
  e = edge_index.shape[1]
  edge32 = edge_index.astype(jnp.int32)

  nblk = -(-e // (NW * B))
  nblk = -(-nblk // NBUF) * NBUF
  e_pad = NW * nblk * B
  pad = e_pad - e
  p_rows = -(-(n_nodes + 1) // (8 * NS)) * (8 * NS)
  pad_idx = np.arange(max(pad, 1), dtype=np.int32)
  row_pad = jnp.asarray(n_nodes + pad_idx % (p_rows - n_nodes))
  col_pad = jnp.asarray(pad_idx % n_nodes)

  sc_k, p_rows2 = _sc_scatter_gather(n_nodes, e, nblk)
  assert p_rows2 == p_rows
  zeros = jnp.zeros((p_rows // NS, D), jnp.float32)
  partials = sc_k(x, edge32, row_pad, col_pad, zeros)
  return _tc_combine(partials, n_nodes)
